# baseline (speedup 1.0000x reference)
import functools

import jax
import jax.numpy as jnp
from jax.experimental import pallas as pl
from jax.experimental.pallas import tpu as pltpu

_B, _T, _DA, _DT, _V = 16, 8, 768, 512, 49408
_N = _B * _T
_TAU = 0.1
_VB = 2048
_NBF = 49152 // _VB
_VMAIN = _NBF * _VB
_VTAIL = _V - _VMAIN


def _sweep_stats_kernel(emb_ref, sum_ref, sq_ref):
    i = pl.program_id(0)
    e = emb_ref[...]
    ee = e * e
    ps = jnp.sum(e.reshape(_VB // 32, 32, _DT), axis=0)
    pq = jnp.sum(ee.reshape(_VB // 32, 32, _DT), axis=0)
    first = i == 0
    sum_ref[...] = jnp.where(first, ps, sum_ref[...] + ps)
    sq_ref[...] = jnp.where(first, pq, sq_ref[...] + pq)


def _finalize_kernel(et_ref, audio_ref, w_ref, b_ref, sum_ref, sq_ref,
                     fn_ref, fnbf_ref):
    et = et_ref[...]
    emb_sum = (jnp.sum(sum_ref[...], axis=0, keepdims=True)
               + jnp.sum(et, axis=0, keepdims=True))
    emb_sq = (jnp.sum(sq_ref[...], axis=0, keepdims=True)
              + jnp.sum(et * et, axis=0, keepdims=True))
    emb_mean = emb_sum / _V
    emb_var = emb_sq / _V - emb_mean * emb_mean
    emb_std = jnp.sqrt(jnp.maximum(emb_var, 0.0))
    feats = (
        jax.lax.dot_general(
            audio_ref[...], w_ref[...], (((1,), (0,)), ((), ())),
            preferred_element_type=jnp.float32,
            precision=jax.lax.Precision.HIGHEST,
        )
        + b_ref[...]
    )
    mu = jnp.mean(feats, axis=0, keepdims=True)
    var = jnp.mean((feats - mu) * (feats - mu), axis=0, keepdims=True)
    normed = (feats - mu) * jax.lax.rsqrt(var + 1e-5)
    f = normed * emb_std + emb_mean
    norm = jnp.sqrt(jnp.sum(f * f, axis=1, keepdims=True)) + 1e-8
    fn = f / norm
    fn_ref[...] = fn
    fnbf_ref[...] = fn.astype(jnp.bfloat16)


def _sweep_score_kernel(fnbf_ref, emb_ref, cos_ref, acc_ref, l_ref):
    i = pl.program_id(0)
    e_bf = emb_ref[...].astype(jnp.bfloat16)
    ee_bf = e_bf * e_bf
    ones_row = jnp.full((1, _DT), 1.0, dtype=jnp.bfloat16)
    rowsq = jax.lax.dot_general(
        ones_row, ee_bf, (((1,), (1,)), ((), ())),
        preferred_element_type=jnp.float32,
    )
    rinv = jax.lax.rsqrt(jnp.maximum(rowsq, 1e-30))
    s = jax.lax.dot_general(
        fnbf_ref[...], e_bf, (((1,), (1,)), ((), ())),
        preferred_element_type=jnp.float32,
    ) * rinv
    cos_ref[...] = s
    p = jnp.exp(s * (1.0 / _TAU))
    p_bf = p.astype(jnp.bfloat16)
    pa = jax.lax.dot_general(
        p_bf, e_bf, (((1,), (0,)), ((), ())),
        preferred_element_type=jnp.float32,
    )
    pl_sum = jnp.sum(p, axis=1, keepdims=True)
    first = i == 0
    acc_ref[...] = jnp.where(first, pa, acc_ref[...] + pa)
    l_ref[...] = jnp.where(first, pl_sum, l_ref[...] + pl_sum)


def _tail_kernel(fn_ref, et_ref, acc_ref, l_ref, _cos_in_ref,
                 cos_ref, kw_ref):
    et = et_ref[...]
    rowsq = jnp.sum(et * et, axis=1, keepdims=True)
    en = et * jax.lax.rsqrt(jnp.maximum(rowsq, 1e-30))
    s = jax.lax.dot_general(
        fn_ref[...], en, (((1,), (1,)), ((), ())),
        preferred_element_type=jnp.float32,
        precision=jax.lax.Precision.HIGHEST,
    )
    cos_ref[...] = s
    p = jnp.exp(s * (1.0 / _TAU))
    acc = acc_ref[...] + jax.lax.dot_general(
        p, et, (((1,), (0,)), ((), ())),
        preferred_element_type=jnp.float32,
        precision=jax.lax.Precision.HIGHEST,
    )
    l = l_ref[...] + jnp.sum(p, axis=1, keepdims=True)
    kw_ref[...] = acc / l


@functools.partial(jax.jit, static_argnames=("interpret",))
def _run(audio_feat, W_proj, b_proj, token_embedding, interpret=False):
    audio2d = audio_feat.reshape(_N, _DA)
    b2d = b_proj.reshape(1, _DT)
    emb_tail = jax.lax.slice(token_embedding, (_VMAIN, 0), (_V, _DT))

    sum32, sq32 = pl.pallas_call(
        _sweep_stats_kernel,
        grid=(_NBF,),
        in_specs=[pl.BlockSpec((_VB, _DT), lambda i: (i, 0))],
        out_specs=[
            pl.BlockSpec((32, _DT), lambda i: (0, 0)),
            pl.BlockSpec((32, _DT), lambda i: (0, 0)),
        ],
        out_shape=[
            jax.ShapeDtypeStruct((32, _DT), jnp.float32),
            jax.ShapeDtypeStruct((32, _DT), jnp.float32),
        ],
        compiler_params=pltpu.CompilerParams(
            dimension_semantics=("arbitrary",),
        ),
        interpret=interpret,
    )(token_embedding)

    fn, fn_bf = pl.pallas_call(
        _finalize_kernel,
        out_shape=[
            jax.ShapeDtypeStruct((_N, _DT), jnp.float32),
            jax.ShapeDtypeStruct((_N, _DT), jnp.bfloat16),
        ],
        interpret=interpret,
    )(emb_tail, audio2d, W_proj, b2d, sum32, sq32)

    cos_main, acc, lsum = pl.pallas_call(
        _sweep_score_kernel,
        grid=(_NBF,),
        in_specs=[
            pl.BlockSpec((_N, _DT), lambda i: (0, 0)),
            pl.BlockSpec((_VB, _DT), lambda i: (i, 0)),
        ],
        out_specs=[
            pl.BlockSpec((_N, _VB), lambda i: (0, i)),
            pl.BlockSpec((_N, _DT), lambda i: (0, 0)),
            pl.BlockSpec((_N, 1), lambda i: (0, 0)),
        ],
        out_shape=[
            jax.ShapeDtypeStruct((_N, _V), jnp.float32),
            jax.ShapeDtypeStruct((_N, _DT), jnp.float32),
            jax.ShapeDtypeStruct((_N, 1), jnp.float32),
        ],
        compiler_params=pltpu.CompilerParams(
            dimension_semantics=("arbitrary",),
        ),
        interpret=interpret,
    )(fn_bf, token_embedding)

    cos, kw = pl.pallas_call(
        _tail_kernel,
        grid=(1,),
        in_specs=[
            pl.BlockSpec((_N, _DT), lambda i: (0, 0)),
            pl.BlockSpec((_VTAIL, _DT), lambda i: (0, 0)),
            pl.BlockSpec((_N, _DT), lambda i: (0, 0)),
            pl.BlockSpec((_N, 1), lambda i: (0, 0)),
            pl.BlockSpec((_N, _VTAIL), lambda i: (0, _VMAIN // _VTAIL)),
        ],
        out_specs=[
            pl.BlockSpec((_N, _VTAIL), lambda i: (0, _VMAIN // _VTAIL)),
            pl.BlockSpec((_N, _DT), lambda i: (0, 0)),
        ],
        out_shape=[
            jax.ShapeDtypeStruct((_N, _V), jnp.float32),
            jax.ShapeDtypeStruct((_N, _DT), jnp.float32),
        ],
        input_output_aliases={4: 0},
        interpret=interpret,
    )(fn, emb_tail, acc, lsum, cos_main)

    keywords = kw.reshape(_B, _T, _DT)
    cos_score = cos.reshape(_B, _T, _V)
    return keywords, cos_score


def kernel(audio_feat, W_proj, b_proj, token_embedding):
    return _run(audio_feat, W_proj, b_proj, token_embedding)


# speedup vs baseline: 1.1110x; 1.1110x over previous
import functools

import jax
import jax.numpy as jnp
from jax.experimental import pallas as pl
from jax.experimental.pallas import tpu as pltpu

_B, _T, _DA, _DT, _V = 16, 8, 768, 512, 49408
_N = _B * _T
_TAU = 0.1
_VB = 4096
_NBF = 49152 // _VB
_VMAIN = _NBF * _VB
_VTAIL = _V - _VMAIN


def _sweep_stats_kernel(emb_ref, sum_ref, sq_ref):
    i = pl.program_id(0)
    e = emb_ref[...]
    ee = e * e
    ps = jnp.sum(e.reshape(_VB // 32, 32, _DT), axis=0)
    pq = jnp.sum(ee.reshape(_VB // 32, 32, _DT), axis=0)
    first = i == 0
    sum_ref[...] = jnp.where(first, ps, sum_ref[...] + ps)
    sq_ref[...] = jnp.where(first, pq, sq_ref[...] + pq)


def _finalize_kernel(et_ref, audio_ref, w_ref, b_ref, sum_ref, sq_ref,
                     fn_ref, fnbf_ref):
    et = et_ref[...]
    emb_sum = (jnp.sum(sum_ref[...], axis=0, keepdims=True)
               + jnp.sum(et, axis=0, keepdims=True))
    emb_sq = (jnp.sum(sq_ref[...], axis=0, keepdims=True)
              + jnp.sum(et * et, axis=0, keepdims=True))
    emb_mean = emb_sum / _V
    emb_var = emb_sq / _V - emb_mean * emb_mean
    emb_std = jnp.sqrt(jnp.maximum(emb_var, 0.0))
    feats = (
        jax.lax.dot_general(
            audio_ref[...], w_ref[...], (((1,), (0,)), ((), ())),
            preferred_element_type=jnp.float32,
            precision=jax.lax.Precision.HIGHEST,
        )
        + b_ref[...]
    )
    mu = jnp.mean(feats, axis=0, keepdims=True)
    var = jnp.mean((feats - mu) * (feats - mu), axis=0, keepdims=True)
    normed = (feats - mu) * jax.lax.rsqrt(var + 1e-5)
    f = normed * emb_std + emb_mean
    norm = jnp.sqrt(jnp.sum(f * f, axis=1, keepdims=True)) + 1e-8
    fn = f / norm
    fn_ref[...] = fn
    fnbf_ref[...] = fn.astype(jnp.bfloat16)


def _sweep_score_kernel(fnbf_ref, emb_ref, cos_ref, acc_ref, l_ref):
    i = pl.program_id(0)
    e_bf = emb_ref[...].astype(jnp.bfloat16)
    ee_bf = e_bf * e_bf
    ones_row = jnp.full((1, _DT), 1.0, dtype=jnp.bfloat16)
    rowsq = jax.lax.dot_general(
        ones_row, ee_bf, (((1,), (1,)), ((), ())),
        preferred_element_type=jnp.float32,
    )
    rinv = jax.lax.rsqrt(jnp.maximum(rowsq, 1e-30))
    s = jax.lax.dot_general(
        fnbf_ref[...], e_bf, (((1,), (1,)), ((), ())),
        preferred_element_type=jnp.float32,
    ) * rinv
    cos_ref[...] = s
    p = jnp.exp(s * (1.0 / _TAU))
    p_bf = p.astype(jnp.bfloat16)
    pa = jax.lax.dot_general(
        p_bf, e_bf, (((1,), (0,)), ((), ())),
        preferred_element_type=jnp.float32,
    )
    pl_sum = jnp.sum(p, axis=1, keepdims=True)
    first = i == 0
    acc_ref[...] = jnp.where(first, pa, acc_ref[...] + pa)
    l_ref[...] = jnp.where(first, pl_sum, l_ref[...] + pl_sum)


def _tail_kernel(fn_ref, et_ref, acc_ref, l_ref, _cos_in_ref,
                 cos_ref, kw_ref):
    et = et_ref[...]
    rowsq = jnp.sum(et * et, axis=1, keepdims=True)
    en = et * jax.lax.rsqrt(jnp.maximum(rowsq, 1e-30))
    s = jax.lax.dot_general(
        fn_ref[...], en, (((1,), (1,)), ((), ())),
        preferred_element_type=jnp.float32,
        precision=jax.lax.Precision.HIGHEST,
    )
    cos_ref[...] = s
    p = jnp.exp(s * (1.0 / _TAU))
    acc = acc_ref[...] + jax.lax.dot_general(
        p, et, (((1,), (0,)), ((), ())),
        preferred_element_type=jnp.float32,
        precision=jax.lax.Precision.HIGHEST,
    )
    l = l_ref[...] + jnp.sum(p, axis=1, keepdims=True)
    kw_ref[...] = acc / l


@functools.partial(jax.jit, static_argnames=("interpret",))
def _run(audio_feat, W_proj, b_proj, token_embedding, interpret=False):
    audio2d = audio_feat.reshape(_N, _DA)
    b2d = b_proj.reshape(1, _DT)
    emb_tail = jax.lax.slice(token_embedding, (_VMAIN, 0), (_V, _DT))

    sum32, sq32 = pl.pallas_call(
        _sweep_stats_kernel,
        grid=(_NBF,),
        in_specs=[pl.BlockSpec((_VB, _DT), lambda i: (i, 0))],
        out_specs=[
            pl.BlockSpec((32, _DT), lambda i: (0, 0)),
            pl.BlockSpec((32, _DT), lambda i: (0, 0)),
        ],
        out_shape=[
            jax.ShapeDtypeStruct((32, _DT), jnp.float32),
            jax.ShapeDtypeStruct((32, _DT), jnp.float32),
        ],
        compiler_params=pltpu.CompilerParams(
            dimension_semantics=("arbitrary",),
        ),
        interpret=interpret,
    )(token_embedding)

    fn, fn_bf = pl.pallas_call(
        _finalize_kernel,
        out_shape=[
            jax.ShapeDtypeStruct((_N, _DT), jnp.float32),
            jax.ShapeDtypeStruct((_N, _DT), jnp.bfloat16),
        ],
        interpret=interpret,
    )(emb_tail, audio2d, W_proj, b2d, sum32, sq32)

    cos_main, acc, lsum = pl.pallas_call(
        _sweep_score_kernel,
        grid=(_NBF,),
        in_specs=[
            pl.BlockSpec((_N, _DT), lambda i: (0, 0)),
            pl.BlockSpec((_VB, _DT), lambda i: (i, 0)),
        ],
        out_specs=[
            pl.BlockSpec((_N, _VB), lambda i: (0, i)),
            pl.BlockSpec((_N, _DT), lambda i: (0, 0)),
            pl.BlockSpec((_N, 1), lambda i: (0, 0)),
        ],
        out_shape=[
            jax.ShapeDtypeStruct((_N, _V), jnp.float32),
            jax.ShapeDtypeStruct((_N, _DT), jnp.float32),
            jax.ShapeDtypeStruct((_N, 1), jnp.float32),
        ],
        compiler_params=pltpu.CompilerParams(
            dimension_semantics=("arbitrary",),
        ),
        interpret=interpret,
    )(fn_bf, token_embedding)

    cos, kw = pl.pallas_call(
        _tail_kernel,
        grid=(1,),
        in_specs=[
            pl.BlockSpec((_N, _DT), lambda i: (0, 0)),
            pl.BlockSpec((_VTAIL, _DT), lambda i: (0, 0)),
            pl.BlockSpec((_N, _DT), lambda i: (0, 0)),
            pl.BlockSpec((_N, 1), lambda i: (0, 0)),
            pl.BlockSpec((_N, _VTAIL), lambda i: (0, _VMAIN // _VTAIL)),
        ],
        out_specs=[
            pl.BlockSpec((_N, _VTAIL), lambda i: (0, _VMAIN // _VTAIL)),
            pl.BlockSpec((_N, _DT), lambda i: (0, 0)),
        ],
        out_shape=[
            jax.ShapeDtypeStruct((_N, _V), jnp.float32),
            jax.ShapeDtypeStruct((_N, _DT), jnp.float32),
        ],
        input_output_aliases={4: 0},
        interpret=interpret,
    )(fn, emb_tail, acc, lsum, cos_main)

    keywords = kw.reshape(_B, _T, _DT)
    cos_score = cos.reshape(_B, _T, _V)
    return keywords, cos_score


def kernel(audio_feat, W_proj, b_proj, token_embedding):
    return _run(audio_feat, W_proj, b_proj, token_embedding)


# speedup vs baseline: 1.1139x; 1.0026x over previous
import functools

import jax
import jax.numpy as jnp
from jax.experimental import pallas as pl
from jax.experimental.pallas import tpu as pltpu

_B, _T, _DA, _DT, _V = 16, 8, 768, 512, 49408
_N = _B * _T
_TAU = 0.1
_VB = 8192
_NBF = 49152 // _VB
_VMAIN = _NBF * _VB
_VTAIL = _V - _VMAIN


def _sweep_stats_kernel(emb_ref, sum_ref, sq_ref):
    i = pl.program_id(0)
    e = emb_ref[...]
    ee = e * e
    ps = jnp.sum(e.reshape(_VB // 32, 32, _DT), axis=0)
    pq = jnp.sum(ee.reshape(_VB // 32, 32, _DT), axis=0)
    first = i == 0
    sum_ref[...] = jnp.where(first, ps, sum_ref[...] + ps)
    sq_ref[...] = jnp.where(first, pq, sq_ref[...] + pq)


def _finalize_kernel(et_ref, audio_ref, w_ref, b_ref, sum_ref, sq_ref,
                     fn_ref, fnbf_ref):
    et = et_ref[...]
    emb_sum = (jnp.sum(sum_ref[...], axis=0, keepdims=True)
               + jnp.sum(et, axis=0, keepdims=True))
    emb_sq = (jnp.sum(sq_ref[...], axis=0, keepdims=True)
              + jnp.sum(et * et, axis=0, keepdims=True))
    emb_mean = emb_sum / _V
    emb_var = emb_sq / _V - emb_mean * emb_mean
    emb_std = jnp.sqrt(jnp.maximum(emb_var, 0.0))
    feats = (
        jax.lax.dot_general(
            audio_ref[...], w_ref[...], (((1,), (0,)), ((), ())),
            preferred_element_type=jnp.float32,
            precision=jax.lax.Precision.HIGHEST,
        )
        + b_ref[...]
    )
    mu = jnp.mean(feats, axis=0, keepdims=True)
    var = jnp.mean((feats - mu) * (feats - mu), axis=0, keepdims=True)
    normed = (feats - mu) * jax.lax.rsqrt(var + 1e-5)
    f = normed * emb_std + emb_mean
    norm = jnp.sqrt(jnp.sum(f * f, axis=1, keepdims=True)) + 1e-8
    fn = f / norm
    fn_ref[...] = fn
    fnbf_ref[...] = fn.astype(jnp.bfloat16)


def _sweep_score_kernel(fnbf_ref, emb_ref, cos_ref, acc_ref, l_ref):
    i = pl.program_id(0)
    e_bf = emb_ref[...].astype(jnp.bfloat16)
    ee_bf = e_bf * e_bf
    ones_row = jnp.full((1, _DT), 1.0, dtype=jnp.bfloat16)
    rowsq = jax.lax.dot_general(
        ones_row, ee_bf, (((1,), (1,)), ((), ())),
        preferred_element_type=jnp.float32,
    )
    rinv = jax.lax.rsqrt(jnp.maximum(rowsq, 1e-30))
    s = jax.lax.dot_general(
        fnbf_ref[...], e_bf, (((1,), (1,)), ((), ())),
        preferred_element_type=jnp.float32,
    ) * rinv
    cos_ref[...] = s
    p = jnp.exp(s * (1.0 / _TAU))
    p_bf = p.astype(jnp.bfloat16)
    pa = jax.lax.dot_general(
        p_bf, e_bf, (((1,), (0,)), ((), ())),
        preferred_element_type=jnp.float32,
    )
    pl_sum = jnp.sum(p, axis=1, keepdims=True)
    first = i == 0
    acc_ref[...] = jnp.where(first, pa, acc_ref[...] + pa)
    l_ref[...] = jnp.where(first, pl_sum, l_ref[...] + pl_sum)


def _tail_kernel(fn_ref, et_ref, acc_ref, l_ref, _cos_in_ref,
                 cos_ref, kw_ref):
    et = et_ref[...]
    rowsq = jnp.sum(et * et, axis=1, keepdims=True)
    en = et * jax.lax.rsqrt(jnp.maximum(rowsq, 1e-30))
    s = jax.lax.dot_general(
        fn_ref[...], en, (((1,), (1,)), ((), ())),
        preferred_element_type=jnp.float32,
        precision=jax.lax.Precision.HIGHEST,
    )
    cos_ref[...] = s
    p = jnp.exp(s * (1.0 / _TAU))
    acc = acc_ref[...] + jax.lax.dot_general(
        p, et, (((1,), (0,)), ((), ())),
        preferred_element_type=jnp.float32,
        precision=jax.lax.Precision.HIGHEST,
    )
    l = l_ref[...] + jnp.sum(p, axis=1, keepdims=True)
    kw_ref[...] = acc / l


@functools.partial(jax.jit, static_argnames=("interpret",))
def _run(audio_feat, W_proj, b_proj, token_embedding, interpret=False):
    audio2d = audio_feat.reshape(_N, _DA)
    b2d = b_proj.reshape(1, _DT)
    emb_tail = jax.lax.slice(token_embedding, (_VMAIN, 0), (_V, _DT))

    sum32, sq32 = pl.pallas_call(
        _sweep_stats_kernel,
        grid=(_NBF,),
        in_specs=[pl.BlockSpec((_VB, _DT), lambda i: (i, 0))],
        out_specs=[
            pl.BlockSpec((32, _DT), lambda i: (0, 0)),
            pl.BlockSpec((32, _DT), lambda i: (0, 0)),
        ],
        out_shape=[
            jax.ShapeDtypeStruct((32, _DT), jnp.float32),
            jax.ShapeDtypeStruct((32, _DT), jnp.float32),
        ],
        compiler_params=pltpu.CompilerParams(
            dimension_semantics=("arbitrary",),
        ),
        interpret=interpret,
    )(token_embedding)

    fn, fn_bf = pl.pallas_call(
        _finalize_kernel,
        out_shape=[
            jax.ShapeDtypeStruct((_N, _DT), jnp.float32),
            jax.ShapeDtypeStruct((_N, _DT), jnp.bfloat16),
        ],
        interpret=interpret,
    )(emb_tail, audio2d, W_proj, b2d, sum32, sq32)

    cos_main, acc, lsum = pl.pallas_call(
        _sweep_score_kernel,
        grid=(_NBF,),
        in_specs=[
            pl.BlockSpec((_N, _DT), lambda i: (0, 0)),
            pl.BlockSpec((_VB, _DT), lambda i: (i, 0)),
        ],
        out_specs=[
            pl.BlockSpec((_N, _VB), lambda i: (0, i)),
            pl.BlockSpec((_N, _DT), lambda i: (0, 0)),
            pl.BlockSpec((_N, 1), lambda i: (0, 0)),
        ],
        out_shape=[
            jax.ShapeDtypeStruct((_N, _V), jnp.float32),
            jax.ShapeDtypeStruct((_N, _DT), jnp.float32),
            jax.ShapeDtypeStruct((_N, 1), jnp.float32),
        ],
        compiler_params=pltpu.CompilerParams(
            dimension_semantics=("arbitrary",),
        ),
        interpret=interpret,
    )(fn_bf, token_embedding)

    cos, kw = pl.pallas_call(
        _tail_kernel,
        grid=(1,),
        in_specs=[
            pl.BlockSpec((_N, _DT), lambda i: (0, 0)),
            pl.BlockSpec((_VTAIL, _DT), lambda i: (0, 0)),
            pl.BlockSpec((_N, _DT), lambda i: (0, 0)),
            pl.BlockSpec((_N, 1), lambda i: (0, 0)),
            pl.BlockSpec((_N, _VTAIL), lambda i: (0, _VMAIN // _VTAIL)),
        ],
        out_specs=[
            pl.BlockSpec((_N, _VTAIL), lambda i: (0, _VMAIN // _VTAIL)),
            pl.BlockSpec((_N, _DT), lambda i: (0, 0)),
        ],
        out_shape=[
            jax.ShapeDtypeStruct((_N, _V), jnp.float32),
            jax.ShapeDtypeStruct((_N, _DT), jnp.float32),
        ],
        input_output_aliases={4: 0},
        interpret=interpret,
    )(fn, emb_tail, acc, lsum, cos_main)

    keywords = kw.reshape(_B, _T, _DT)
    cos_score = cos.reshape(_B, _T, _V)
    return keywords, cos_score


def kernel(audio_feat, W_proj, b_proj, token_embedding):
    return _run(audio_feat, W_proj, b_proj, token_embedding)


# speedup vs baseline: 1.1339x; 1.0179x over previous
import functools

import jax
import jax.numpy as jnp
from jax.experimental import pallas as pl
from jax.experimental.pallas import tpu as pltpu

_B, _T, _DA, _DT, _V = 16, 8, 768, 512, 49408
_N = _B * _T
_TAU = 0.1
_VB = 4096
_NBF = 49152 // _VB
_VMAIN = _NBF * _VB
_VTAIL = _V - _VMAIN


def _sweep_stats_kernel(emb_ref, sum_ref, sq_ref):
    i = pl.program_id(0)
    e = emb_ref[...]
    ee = e * e
    ps = jnp.sum(e.reshape(_VB // 32, 32, _DT), axis=0)
    pq = jnp.sum(ee.reshape(_VB // 32, 32, _DT), axis=0)
    first = i == 0
    sum_ref[...] = jnp.where(first, ps, sum_ref[...] + ps)
    sq_ref[...] = jnp.where(first, pq, sq_ref[...] + pq)


def _finalize_kernel(et_ref, audio_ref, w_ref, b_ref, sum_ref, sq_ref,
                     fn_ref, fnbf_ref):
    et = et_ref[...]
    emb_sum = (jnp.sum(sum_ref[...], axis=0, keepdims=True)
               + jnp.sum(et, axis=0, keepdims=True))
    emb_sq = (jnp.sum(sq_ref[...], axis=0, keepdims=True)
              + jnp.sum(et * et, axis=0, keepdims=True))
    emb_mean = emb_sum / _V
    emb_var = emb_sq / _V - emb_mean * emb_mean
    emb_std = jnp.sqrt(jnp.maximum(emb_var, 0.0))
    feats = (
        jax.lax.dot_general(
            audio_ref[...], w_ref[...], (((1,), (0,)), ((), ())),
            preferred_element_type=jnp.float32,
            precision=jax.lax.Precision.HIGHEST,
        )
        + b_ref[...]
    )
    mu = jnp.mean(feats, axis=0, keepdims=True)
    var = jnp.mean((feats - mu) * (feats - mu), axis=0, keepdims=True)
    normed = (feats - mu) * jax.lax.rsqrt(var + 1e-5)
    f = normed * emb_std + emb_mean
    norm = jnp.sqrt(jnp.sum(f * f, axis=1, keepdims=True)) + 1e-8
    fn = f / norm
    fn_ref[...] = fn
    fnbf_ref[...] = fn.astype(jnp.bfloat16)


def _sweep_score_kernel(fnbf_ref, emb_ref, cos_ref, acc_ref, l_ref):
    i = pl.program_id(0)
    e_bf = emb_ref[...].astype(jnp.bfloat16)
    ee_bf = e_bf * e_bf
    ones_row = jnp.full((1, _DT), 1.0, dtype=jnp.bfloat16)
    rowsq = jax.lax.dot_general(
        ones_row, ee_bf, (((1,), (1,)), ((), ())),
        preferred_element_type=jnp.float32,
    )
    rinv = jax.lax.rsqrt(jnp.maximum(rowsq, 1e-30))
    s = jax.lax.dot_general(
        fnbf_ref[...], e_bf, (((1,), (1,)), ((), ())),
        preferred_element_type=jnp.float32,
    ) * rinv
    cos_ref[...] = s
    p = jnp.exp(s * (1.0 / _TAU))
    p_bf = p.astype(jnp.bfloat16)
    pa = jax.lax.dot_general(
        p_bf, e_bf, (((1,), (0,)), ((), ())),
        preferred_element_type=jnp.float32,
    )
    pl_sum = jnp.sum(p, axis=1, keepdims=True)
    first = i == 0
    acc_ref[...] = jnp.where(first, pa, acc_ref[...] + pa)
    l_ref[...] = jnp.where(first, pl_sum, l_ref[...] + pl_sum)


def _tail_kernel(fn_ref, et_ref, acc_ref, l_ref, _cos_in_ref,
                 cos_ref, kw_ref):
    et = et_ref[...]
    rowsq = jnp.sum(et * et, axis=1, keepdims=True)
    en = et * jax.lax.rsqrt(jnp.maximum(rowsq, 1e-30))
    s = jax.lax.dot_general(
        fn_ref[...], en, (((1,), (1,)), ((), ())),
        preferred_element_type=jnp.float32,
        precision=jax.lax.Precision.HIGHEST,
    )
    cos_ref[...] = s
    p = jnp.exp(s * (1.0 / _TAU))
    acc = acc_ref[...] + jax.lax.dot_general(
        p, et, (((1,), (0,)), ((), ())),
        preferred_element_type=jnp.float32,
        precision=jax.lax.Precision.HIGHEST,
    )
    l = l_ref[...] + jnp.sum(p, axis=1, keepdims=True)
    kw_ref[...] = acc / l


@functools.partial(jax.jit, static_argnames=("interpret",))
def _run(audio_feat, W_proj, b_proj, token_embedding, interpret=False):
    audio2d = audio_feat.reshape(_N, _DA)
    b2d = b_proj.reshape(1, _DT)

    sum32, sq32 = pl.pallas_call(
        _sweep_stats_kernel,
        grid=(_NBF,),
        in_specs=[pl.BlockSpec((_VB, _DT), lambda i: (i, 0))],
        out_specs=[
            pl.BlockSpec((32, _DT), lambda i: (0, 0)),
            pl.BlockSpec((32, _DT), lambda i: (0, 0)),
        ],
        out_shape=[
            jax.ShapeDtypeStruct((32, _DT), jnp.float32),
            jax.ShapeDtypeStruct((32, _DT), jnp.float32),
        ],
        compiler_params=pltpu.CompilerParams(
            dimension_semantics=("arbitrary",),
        ),
        interpret=interpret,
    )(token_embedding)

    fn, fn_bf = pl.pallas_call(
        _finalize_kernel,
        grid=(1,),
        in_specs=[
            pl.BlockSpec((_VTAIL, _DT), lambda i: (_VMAIN // _VTAIL, 0)),
            pl.BlockSpec((_N, _DA), lambda i: (0, 0)),
            pl.BlockSpec((_DA, _DT), lambda i: (0, 0)),
            pl.BlockSpec((1, _DT), lambda i: (0, 0)),
            pl.BlockSpec((32, _DT), lambda i: (0, 0)),
            pl.BlockSpec((32, _DT), lambda i: (0, 0)),
        ],
        out_specs=[
            pl.BlockSpec((_N, _DT), lambda i: (0, 0)),
            pl.BlockSpec((_N, _DT), lambda i: (0, 0)),
        ],
        out_shape=[
            jax.ShapeDtypeStruct((_N, _DT), jnp.float32),
            jax.ShapeDtypeStruct((_N, _DT), jnp.bfloat16),
        ],
        interpret=interpret,
    )(token_embedding, audio2d, W_proj, b2d, sum32, sq32)

    cos_main, acc, lsum = pl.pallas_call(
        _sweep_score_kernel,
        grid=(_NBF,),
        in_specs=[
            pl.BlockSpec((_N, _DT), lambda i: (0, 0)),
            pl.BlockSpec((_VB, _DT), lambda i: (i, 0)),
        ],
        out_specs=[
            pl.BlockSpec((_N, _VB), lambda i: (0, i)),
            pl.BlockSpec((_N, _DT), lambda i: (0, 0)),
            pl.BlockSpec((_N, 1), lambda i: (0, 0)),
        ],
        out_shape=[
            jax.ShapeDtypeStruct((_N, _V), jnp.float32),
            jax.ShapeDtypeStruct((_N, _DT), jnp.float32),
            jax.ShapeDtypeStruct((_N, 1), jnp.float32),
        ],
        compiler_params=pltpu.CompilerParams(
            dimension_semantics=("arbitrary",),
        ),
        interpret=interpret,
    )(fn_bf, token_embedding)

    cos, kw = pl.pallas_call(
        _tail_kernel,
        grid=(1,),
        in_specs=[
            pl.BlockSpec((_N, _DT), lambda i: (0, 0)),
            pl.BlockSpec((_VTAIL, _DT), lambda i: (_VMAIN // _VTAIL, 0)),
            pl.BlockSpec((_N, _DT), lambda i: (0, 0)),
            pl.BlockSpec((_N, 1), lambda i: (0, 0)),
            pl.BlockSpec((_N, _VTAIL), lambda i: (0, _VMAIN // _VTAIL)),
        ],
        out_specs=[
            pl.BlockSpec((_N, _VTAIL), lambda i: (0, _VMAIN // _VTAIL)),
            pl.BlockSpec((_N, _DT), lambda i: (0, 0)),
        ],
        out_shape=[
            jax.ShapeDtypeStruct((_N, _V), jnp.float32),
            jax.ShapeDtypeStruct((_N, _DT), jnp.float32),
        ],
        input_output_aliases={4: 0},
        interpret=interpret,
    )(fn, token_embedding, acc, lsum, cos_main)

    keywords = kw.reshape(_B, _T, _DT)
    cos_score = cos.reshape(_B, _T, _V)
    return keywords, cos_score


def kernel(audio_feat, W_proj, b_proj, token_embedding):
    return _run(audio_feat, W_proj, b_proj, token_embedding)
